# baseline (speedup 1.0000x reference)
import functools

import jax
import jax.numpy as jnp
from jax import lax
from jax.experimental import pallas as pl
from jax.experimental.pallas import tpu as pltpu
from jax.experimental.pallas import tpu_sc as plsc

_ROWS = 262144
_DIM = 128
_NC = 2
_NS = 16
_NW = _NC * _NS
_ROWS_W = _ROWS // _NW
_CHUNK = 128
_NBUF = 4
_LAG = 2
_NITER = _ROWS_W // _CHUNK
_NGROUPS = _NITER // _NBUF


def _start_in(in_hbm, buf, in_sems, row, b):
    pltpu.make_async_copy(
        in_hbm.at[pl.ds(row, _CHUNK), :], buf.at[b], in_sems.at[b]
    ).start()


def _wait_in(in_hbm, buf, in_sems, row, b):
    pltpu.make_async_copy(
        in_hbm.at[pl.ds(row, _CHUNK), :], buf.at[b], in_sems.at[b]
    ).wait()


def _start_out(out_hbm, buf, out_sems, row, b):
    pltpu.make_async_copy(
        buf.at[b], out_hbm.at[pl.ds(row, _CHUNK), :], out_sems.at[b]
    ).start()


def _wait_out(out_hbm, buf, out_sems, row, b):
    pltpu.make_async_copy(
        buf.at[b], out_hbm.at[pl.ds(row, _CHUNK), :], out_sems.at[b]
    ).wait()


def _sc_copy_body(in_hbm, out_hbm, buf, in_sems, out_sems):
    wid = lax.axis_index("s") * _NC + lax.axis_index("c")
    base = wid * _ROWS_W

    for b in range(_NBUF):
        _start_in(in_hbm, buf, in_sems, base + b * _CHUNK, b)

    def step(i_dyn, b, do_recycle):
        row = base + i_dyn * _CHUNK
        _wait_in(in_hbm, buf, in_sems, row, b)
        _start_out(out_hbm, buf, out_sems, row, b)
        if do_recycle:
            bt = (b - _LAG) % _NBUF
            trow = row - _LAG * _CHUNK
            _wait_out(out_hbm, buf, out_sems, trow, bt)
            _start_in(in_hbm, buf, in_sems, trow + _NBUF * _CHUNK, bt)

    for b in range(_NBUF):
        step(b, b, do_recycle=(b >= _LAG))

    def group(g, carry):
        for b in range(_NBUF):
            step(g * _NBUF + b, b, do_recycle=True)
        return carry

    lax.fori_loop(1, _NGROUPS - 1, group, 0)

    last = (_NGROUPS - 1) * _NBUF
    for b in range(_NBUF):
        step(last + b, b, do_recycle=(b < _LAG))

    for b in range(_NBUF):
        _wait_out(out_hbm, buf, out_sems, base + (last + b) * _CHUNK, b)


_sc_copy = functools.partial(
    pl.kernel,
    out_type=jax.ShapeDtypeStruct((_ROWS, _DIM), jnp.float32),
    mesh=plsc.VectorSubcoreMesh(core_axis_name="c", subcore_axis_name="s"),
    scratch_types=[
        pltpu.VMEM((_NBUF, _CHUNK, _DIM), jnp.float32),
        pltpu.SemaphoreType.DMA((_NBUF,)),
        pltpu.SemaphoreType.DMA((_NBUF,)),
    ],
)(_sc_copy_body)


def kernel(queue):
    return _sc_copy(queue)
